# baseline (speedup 1.0000x reference)
import functools

import jax
import jax.numpy as jnp
from jax import lax
from jax.experimental import pallas as pl
from jax.experimental.pallas import tpu as pltpu
from jax.experimental.pallas import tpu_sc as plsc

NC = 2
NS = 16
L = 16
NW = NC * NS

B = 16384
D = 64
BPW = B // NW
CHUNK = 128
NCHUNK = BPW // CHUNK
NG = BPW // L

_EPS = 1e-8


def _sqrt16(x):
    xg = jnp.maximum(x, 1e-30)
    i = plsc.bitcast(xg, jnp.int32)
    y = plsc.bitcast(jnp.full((L,), 0x5F3759DF, jnp.int32) - (i >> 1),
                     jnp.float32)
    for _ in range(3):
        y = y * (1.5 - 0.5 * xg * y * y)
    return x * y


def _body(pred_ids_hbm, ent_ids_hbm, ent_tab_hbm, pred_tab_hbm, out_hbm,
          eidx_v, pidx_v, erows_v, prows_v, out_v, sem):
    wid = lax.axis_index("s") * NC + lax.axis_index("c")
    base = wid * BPW

    for j in range(NCHUNK):
        pltpu.sync_copy(ent_ids_hbm.at[pl.ds(base + j * CHUNK, CHUNK)],
                        eidx_v.at[j])
        pltpu.sync_copy(pred_ids_hbm.at[pl.ds(base + j * CHUNK, CHUNK)],
                        pidx_v.at[j])

    handles = []
    for j in range(NCHUNK):
        handles.append(pltpu.async_copy(
            ent_tab_hbm.at[eidx_v.at[j]],
            erows_v.at[pl.ds(j * CHUNK, CHUNK)], sem))
        handles.append(pltpu.async_copy(
            pred_tab_hbm.at[pidx_v.at[j]],
            prows_v.at[pl.ds(j * CHUNK, CHUNK)], sem))
    for h in handles:
        h.wait()

    def group(g, carry):
        rows = g * L + lax.iota(jnp.int32, L)
        num = jnp.zeros((L,), jnp.float32)
        e2 = jnp.zeros((L,), jnp.float32)
        p2 = jnp.zeros((L,), jnp.float32)
        for dd in range(D):
            col = jnp.full((L,), dd, jnp.int32)
            e = plsc.load_gather(erows_v, [rows, col])
            p = plsc.load_gather(prows_v, [rows, col])
            num = num + e * p
            e2 = e2 + e * e
            p2 = p2 + p * p
        denom = jnp.maximum(_sqrt16(e2) * _sqrt16(p2), _EPS)
        truth = 0.5 * (num / denom) + 0.5
        out_v[pl.ds(g * L, L)] = truth
        return carry

    lax.fori_loop(0, NG, group, 0)

    pltpu.sync_copy(out_v, out_hbm.at[pl.ds(base, BPW)])


_mesh = plsc.VectorSubcoreMesh(core_axis_name="c", subcore_axis_name="s",
                               num_cores=NC, num_subcores=NS)

_sc_call = pl.kernel(
    _body,
    out_type=jax.ShapeDtypeStruct((B,), jnp.float32),
    mesh=_mesh,
    scratch_types=[
        pltpu.VMEM((NCHUNK, CHUNK), jnp.int32),
        pltpu.VMEM((NCHUNK, CHUNK), jnp.int32),
        pltpu.VMEM((BPW, D), jnp.float32),
        pltpu.VMEM((BPW, D), jnp.float32),
        pltpu.VMEM((BPW,), jnp.float32),
        pltpu.SemaphoreType.DMA,
    ],
    compiler_params=pltpu.CompilerParams(needs_layout_passes=False,
                                         use_tc_tiling_on_sc=False),
)


@jax.jit
def kernel(predicate_ids, entity_ids, entity_table, predicate_table):
    return _sc_call(predicate_ids, entity_ids, entity_table, predicate_table)


# speedup vs baseline: 1.0016x; 1.0016x over previous
import functools

import jax
import jax.numpy as jnp
from jax import lax
from jax.experimental import pallas as pl
from jax.experimental.pallas import tpu as pltpu
from jax.experimental.pallas import tpu_sc as plsc

NC = 2
NS = 16
L = 16
NW = NC * NS

B = 16384
D = 64
BPW = B // NW
CHUNK = 128
NCHUNK = BPW // CHUNK
NG = BPW // L

_EPS = 1e-8


def _sqrt16(x):
    xg = jnp.maximum(x, 1e-30)
    i = plsc.bitcast(xg, jnp.int32)
    y = plsc.bitcast(jnp.full((L,), 0x5F3759DF, jnp.int32) - (i >> 1),
                     jnp.float32)
    for _ in range(3):
        y = y * (1.5 - 0.5 * xg * y * y)
    return x * y


def _body(pred_ids_hbm, ent_ids_hbm, ent_tab_hbm, pred_tab_hbm, out_hbm,
          eidx_v, sidx_v, pidx_v, erows_v, prows_v, out_v, sem):
    wid = lax.axis_index("s") * NC + lax.axis_index("c")
    base = wid * BPW

    c1 = pltpu.async_copy(ent_ids_hbm.at[pl.ds(base, BPW)], eidx_v, sem)
    c2 = pltpu.async_copy(pred_ids_hbm.at[pl.ds(base, BPW)], pidx_v, sem)
    c1.wait()
    c2.wait()

    for j in range(NG):
        sl = pl.ds(j * L, L)
        sidx_v[sl] = eidx_v[sl] >> 1

    handles = []
    for j in range(NCHUNK):
        handles.append(pltpu.async_copy(
            ent_tab_hbm.at[sidx_v.at[pl.ds(j * CHUNK, CHUNK)]],
            erows_v.at[pl.ds(j * CHUNK, CHUNK)], sem))
        handles.append(pltpu.async_copy(
            pred_tab_hbm.at[pidx_v.at[pl.ds(j * CHUNK, CHUNK)]],
            prows_v.at[pl.ds(j * CHUNK, CHUNK)], sem))
    for h in handles:
        h.wait()

    def group(g, carry):
        rows = g * L + lax.iota(jnp.int32, L)
        ecol0 = (eidx_v[pl.ds(g * L, L)] & 1) * D
        num = jnp.zeros((L,), jnp.float32)
        e2 = jnp.zeros((L,), jnp.float32)
        p2 = jnp.zeros((L,), jnp.float32)
        for dd in range(D):
            pcol = jnp.full((L,), dd, jnp.int32)
            e = plsc.load_gather(erows_v, [rows, ecol0 + dd])
            p = plsc.load_gather(prows_v, [rows, pcol])
            num = num + e * p
            e2 = e2 + e * e
            p2 = p2 + p * p
        denom = jnp.maximum(_sqrt16(e2) * _sqrt16(p2), _EPS)
        truth = 0.5 * (num / denom) + 0.5
        out_v[pl.ds(g * L, L)] = truth
        return carry

    lax.fori_loop(0, NG, group, 0)

    pltpu.sync_copy(out_v, out_hbm.at[pl.ds(base, BPW)])


_mesh = plsc.VectorSubcoreMesh(core_axis_name="c", subcore_axis_name="s",
                               num_cores=NC, num_subcores=NS)

_sc_call = pl.kernel(
    _body,
    out_type=jax.ShapeDtypeStruct((B,), jnp.float32),
    mesh=_mesh,
    scratch_types=[
        pltpu.VMEM((BPW,), jnp.int32),
        pltpu.VMEM((BPW,), jnp.int32),
        pltpu.VMEM((BPW,), jnp.int32),
        pltpu.VMEM((BPW, 2 * D), jnp.float32),
        pltpu.VMEM((BPW, D), jnp.float32),
        pltpu.VMEM((BPW,), jnp.float32),
        pltpu.SemaphoreType.DMA,
    ],
    compiler_params=pltpu.CompilerParams(needs_layout_passes=False,
                                         use_tc_tiling_on_sc=False),
)


@jax.jit
def kernel(predicate_ids, entity_ids, entity_table, predicate_table):
    ent2 = entity_table.reshape(entity_table.shape[0] // 2, 2 * D)
    return _sc_call(predicate_ids, entity_ids, ent2, predicate_table)
